# baseline (speedup 1.0000x reference)
import functools
import jax
import jax.numpy as jnp
from jax import lax
from jax.experimental import pallas as pl
from jax.experimental.pallas import tpu as pltpu
from jax.experimental.pallas import tpu_sc as plsc

_NC = 2
_NS = 16
_L = 16
_OG = 8
_BG = 4
_CPB = 4


def _sc_body(f, g, opg, bc, nbv, xc_ref, cre_ref, gvec_ref, bvec_ref,
             out_ref, cv, xbuf, yv, gv, bv):
    wid = lax.axis_index("s") * _NC + lax.axis_index("c")
    og = lax.rem(wid, _OG)
    bg = lax.div(wid, _OG)

    pltpu.sync_copy(cre_ref.at[og], cv)
    pltpu.sync_copy(gvec_ref, gv)
    pltpu.sync_copy(bvec_ref, bv)

    ghead = gv[pl.ds(0, _L)]
    gtail = gv[pl.ds(g - _L, _L)]
    gmin = ghead[0]
    invh = (g - 1.0) / jnp.full((_L,), gtail[_L - 1] - gmin, jnp.float32)
    lanes = lax.iota(jnp.int32, _L)
    xlane = lanes * f
    ob = og * opg
    bias_init = tuple(
        plsc.load_gather(bv, [jnp.full((_L,), ob + j, jnp.int32)])
        for j in range(opg))

    unroll = 4

    def f_body(fu, accs):
        accs = list(accs)
        for s in range(unroll):
            fi = fu * unroll + s
            xv = plsc.load_gather(xbuf, [f_body_base[0] + fi])
            u = (xv - gmin) * invh
            it = u.astype(jnp.int32)
            ic = it + jnp.where(u > it.astype(jnp.float32), 1, 0)
            idx = jnp.clip(ic, 1, g - 1)
            i0 = idx - 1
            t = u - i0.astype(jnp.float32)
            ib0 = i0 * opg + fi * (g * opg)
            ib1 = ib0 + opg
            for j in range(opg):
                c0 = plsc.load_gather(cv, [ib0 + j])
                c1 = plsc.load_gather(cv, [ib1 + j])
                accs[j] = accs[j] + (c0 + t * (c1 - c0))
        return tuple(accs)

    f_body_base = [None]

    def bvec_body(bi, carry):
        bb = bi * _L
        f_body_base[0] = bb * f + xlane
        accs = lax.fori_loop(0, f // unroll, f_body, bias_init)
        for j in range(opg):
            yv[j, pl.ds(bb, _L)] = accs[j]
        return carry

    def chunk_body(ci, carry):
        cg = bg * _CPB + ci
        pltpu.sync_copy(xc_ref.at[cg], xbuf)
        lax.fori_loop(0, bc // _L, bvec_body, 0)
        pltpu.sync_copy(yv, out_ref.at[cg, og])
        return carry

    lax.fori_loop(0, _CPB, chunk_body, 0)


def kernel(x, coeff, bias, grid):
    x = x.astype(jnp.float32)
    if x.ndim != 2:
        x = x.reshape(x.shape[0], -1)
    b, f = x.shape
    g = grid.shape[0]
    o = coeff.shape[-1]
    opg = o // _OG
    nch = _BG * _CPB
    bc = b // nch

    xc = x.reshape(nch, bc * f)
    cre = (coeff.astype(jnp.float32).reshape(f, g, _OG, opg)
           .transpose(2, 0, 1, 3).reshape(_OG, f * g * opg))
    gvec = grid.astype(jnp.float32)
    bvec = bias.astype(jnp.float32)

    mesh = plsc.VectorSubcoreMesh(core_axis_name="c", subcore_axis_name="s")
    run = functools.partial(
        pl.kernel,
        mesh=mesh,
        compiler_params=pltpu.CompilerParams(needs_layout_passes=False),
        out_type=jax.ShapeDtypeStruct((nch, _OG, opg, bc), jnp.float32),
        scratch_types=[
            pltpu.VMEM((f * g * opg,), jnp.float32),
            pltpu.VMEM((bc * f,), jnp.float32),
            pltpu.VMEM((opg, bc), jnp.float32),
            pltpu.VMEM((g,), jnp.float32),
            pltpu.VMEM((o,), jnp.float32),
        ],
    )(functools.partial(_sc_body, f, g, opg, bc, _OG * opg))
    yblk = run(xc, cre, gvec, bvec)
    return yblk.transpose(0, 3, 1, 2).reshape(b, o)


# speedup vs baseline: 2.5044x; 2.5044x over previous
import functools
import jax
import jax.numpy as jnp
from jax import lax
from jax.experimental import pallas as pl
from jax.experimental.pallas import tpu as pltpu
from jax.experimental.pallas import tpu_sc as plsc

_NC = 2
_NS = 16
_L = 16
_OG = 8
_BG = 4
_CPB = 4


def _sc_body(f, g, opg, bc, nbv, xc_ref, cre_ref, gvec_ref, bvec_ref,
             out_ref, cv, xbuf, yv, gv, bv):
    wid = lax.axis_index("s") * _NC + lax.axis_index("c")
    og = lax.rem(wid, _OG)
    bg = lax.div(wid, _OG)

    pltpu.sync_copy(cre_ref.at[og], cv)
    pltpu.sync_copy(gvec_ref, gv)
    pltpu.sync_copy(bvec_ref, bv)

    ghead = gv[pl.ds(0, _L)]
    gtail = gv[pl.ds(g - _L, _L)]
    gmin = ghead[0]
    invh = (g - 1.0) / jnp.full((_L,), gtail[_L - 1] - gmin, jnp.float32)
    ob = og * opg
    bias_init = tuple(
        plsc.load_gather(bv, [jnp.full((_L,), ob + j, jnp.int32)])
        for j in range(opg))

    def f_body(fi, accs):
        xv = xbuf[pl.ds(fi * bc + f_body_base[0], _L)]
        u = (xv - gmin) * invh
        it = u.astype(jnp.int32)
        ic = it + jnp.where(u > it.astype(jnp.float32), 1, 0)
        idx = jnp.clip(ic, 1, g - 1)
        i0 = idx - 1
        t = u - i0.astype(jnp.float32)
        ib0 = i0 + fi * (g * opg)
        out = []
        for j in range(opg):
            c0 = plsc.load_gather(cv, [ib0 + j * g])
            c1 = plsc.load_gather(cv, [ib0 + (j * g + 1)])
            out.append(accs[j] + (c0 + t * (c1 - c0)))
        return tuple(out)

    f_body_base = [None]

    def bvec_body(bi, carry):
        bb = bi * _L
        f_body_base[0] = bb
        accs = lax.fori_loop(0, f, f_body, bias_init)
        for j in range(opg):
            yv[j, pl.ds(bb, _L)] = accs[j]
        return carry

    def chunk_body(ci, carry):
        cg = bg * _CPB + ci
        pltpu.sync_copy(xc_ref.at[cg], xbuf)
        lax.fori_loop(0, bc // _L, bvec_body, 0)
        pltpu.sync_copy(yv, out_ref.at[cg, og])
        return carry

    lax.fori_loop(0, _CPB, chunk_body, 0)


def kernel(x, coeff, bias, grid):
    x = x.astype(jnp.float32)
    if x.ndim != 2:
        x = x.reshape(x.shape[0], -1)
    b, f = x.shape
    g = grid.shape[0]
    o = coeff.shape[-1]
    opg = o // _OG
    nch = _BG * _CPB
    bc = b // nch

    xc = x.reshape(nch, bc, f).transpose(0, 2, 1).reshape(nch, f * bc)
    cre = (coeff.astype(jnp.float32).reshape(f, g, _OG, opg)
           .transpose(2, 0, 3, 1).reshape(_OG, f * opg * g))
    gvec = grid.astype(jnp.float32)
    bvec = bias.astype(jnp.float32)

    mesh = plsc.VectorSubcoreMesh(core_axis_name="c", subcore_axis_name="s")
    run = functools.partial(
        pl.kernel,
        mesh=mesh,
        compiler_params=pltpu.CompilerParams(needs_layout_passes=False),
        out_type=jax.ShapeDtypeStruct((nch, _OG, opg, bc), jnp.float32),
        scratch_types=[
            pltpu.VMEM((f * g * opg,), jnp.float32),
            pltpu.VMEM((bc * f,), jnp.float32),
            pltpu.VMEM((opg, bc), jnp.float32),
            pltpu.VMEM((g,), jnp.float32),
            pltpu.VMEM((o,), jnp.float32),
        ],
    )(functools.partial(_sc_body, f, g, opg, bc, _OG * opg))
    yblk = run(xc, cre, gvec, bvec)
    return yblk.transpose(0, 3, 1, 2).reshape(b, o)


# speedup vs baseline: 3.9510x; 1.5776x over previous
import functools
import jax
import jax.numpy as jnp
import numpy as np
from jax import lax
from jax.experimental import pallas as pl
from jax.experimental.pallas import tpu as pltpu
from jax.experimental.pallas import tpu_sc as plsc

_NC = 2
_NS = 16
_L = 16
_OG = 8
_BG = 4
_CPB = 4


def _sc_body(f, g, opg, bc, nbv, xc_ref, cre_ref, gvec_ref, bvec_ref,
             out_ref, cv, xbuf, yv, gv, bv):
    wid = lax.axis_index("s") * _NC + lax.axis_index("c")
    og = lax.rem(wid, _OG)
    bg = lax.div(wid, _OG)

    pltpu.sync_copy(cre_ref.at[og], cv)
    pltpu.sync_copy(gvec_ref, gv)
    pltpu.sync_copy(bvec_ref, bv)

    ghead = gv[pl.ds(0, _L)]
    gtail = gv[pl.ds(g - _L, _L)]
    gmin = ghead[0]
    invh = (g - 1.0) / jnp.full((_L,), gtail[_L - 1] - gmin, jnp.float32)
    ob = og * opg
    bias_init = tuple(
        plsc.load_gather(bv, [jnp.full((_L,), ob + j, jnp.int32)])
        for j in range(opg))

    def f_body(fi, accs):
        xv = xbuf[pl.ds(fi * bc + f_body_base[0], _L)]
        u = (xv - gmin) * invh
        it = u.astype(jnp.int32)
        ic = it + jnp.where(u > it.astype(jnp.float32), 1, 0)
        idx = jnp.clip(ic, 1, g - 1)
        i0 = idx - 1
        t = u - i0.astype(jnp.float32)
        ib0 = i0 + fi * (g * opg)
        out = []
        for j in range(opg):
            c0 = plsc.load_gather(cv, [ib0 + j * g])
            c1 = plsc.load_gather(cv, [ib0 + (j * g + 1)])
            out.append(accs[j] + (c0 + t * (c1 - c0)))
        return tuple(out)

    f_body_base = [None]

    def bvec_body(bi, carry):
        bb = bi * _L
        f_body_base[0] = bb
        accs = lax.fori_loop(0, f, f_body, bias_init)
        for j in range(opg):
            yv[j, pl.ds(bb, _L)] = accs[j]
        return carry

    def chunk_body(ci, carry):
        cg = bg * _CPB + ci
        pltpu.sync_copy(xc_ref.at[cg], xbuf)
        lax.fori_loop(0, bc // _L, bvec_body, 0)
        pltpu.sync_copy(yv, out_ref.at[cg, og])
        return carry

    lax.fori_loop(0, _CPB, chunk_body, 0)


def _sc_kan(x, coeff, bias, grid):
    b, f = x.shape
    g = grid.shape[0]
    o = coeff.shape[-1]
    opg = o // _OG
    nch = _BG * _CPB
    bc = b // nch

    xc = x.reshape(nch, bc, f).transpose(0, 2, 1).reshape(nch, f * bc)
    cre = (coeff.astype(jnp.float32).reshape(f, g, _OG, opg)
           .transpose(2, 0, 3, 1).reshape(_OG, f * opg * g))
    gvec = grid.astype(jnp.float32)
    bvec = bias.astype(jnp.float32)

    mesh = plsc.VectorSubcoreMesh(core_axis_name="c", subcore_axis_name="s")
    run = functools.partial(
        pl.kernel,
        mesh=mesh,
        compiler_params=pltpu.CompilerParams(needs_layout_passes=False),
        out_type=jax.ShapeDtypeStruct((nch, _OG, opg, bc), jnp.float32),
        scratch_types=[
            pltpu.VMEM((f * g * opg,), jnp.float32),
            pltpu.VMEM((bc * f,), jnp.float32),
            pltpu.VMEM((opg, bc), jnp.float32),
            pltpu.VMEM((g,), jnp.float32),
            pltpu.VMEM((o,), jnp.float32),
        ],
    )(functools.partial(_sc_body, f, g, opg, bc, _OG * opg))
    yblk = run(xc, cre, gvec, bvec)
    return yblk.transpose(0, 3, 1, 2).reshape(b, o)


def _tc_body(nc, fc, g, o, x_ref, gvec_ref, glane_ref, creo_ref, cd_ref,
             bias_ref, o_ref):
    x = x_ref[...]
    gmin = gvec_ref[0, 0]
    gmax = gvec_ref[0, g - 1]
    invh = (g - 1.0) / (gmax - gmin)
    u = (x - gmin) * invh
    uc = jnp.clip(u, 0.0, g - 1.0)
    d = u - uc
    dneg = jnp.minimum(d, 0.0)
    dpos = d - dneg

    glane = glane_ref[...]
    lanes = fc * g
    acc = None
    for c in range(nc):
        ucc = uc[:, c * fc:(c + 1) * fc]
        uce = pltpu.repeat(ucc, g, axis=1)
        dd = (uce - glane).astype(jnp.bfloat16)
        s = jnp.maximum(jnp.bfloat16(1.0) - jnp.abs(dd), jnp.bfloat16(0.0))
        p = jnp.dot(s, creo_ref[c * lanes:(c + 1) * lanes, :],
                    preferred_element_type=jnp.float32)
        acc = p if acc is None else acc + p

    dcat = jnp.concatenate([dneg, dpos], axis=1)
    acc = acc + jnp.dot(dcat, cd_ref[...],
                        preferred_element_type=jnp.float32)
    o_ref[...] = acc + bias_ref[...]


def _tc_kan(x, coeff, bias, grid):
    b, f = x.shape
    g = grid.shape[0]
    o = coeff.shape[-1]

    fc = 16
    nc = f // fc
    lanes = fc * g
    bb = min(b, 1024)

    creo = coeff.astype(jnp.float32).reshape(nc, fc, g, o)
    creo = creo.transpose(0, 2, 1, 3).reshape(nc * g * fc, o)
    creo = creo.astype(jnp.bfloat16)
    cd = jnp.concatenate([coeff[:, 1, :] - coeff[:, 0, :],
                          coeff[:, g - 1, :] - coeff[:, g - 2, :]],
                         axis=0).astype(jnp.float32)
    glane = jnp.asarray(
        np.repeat(np.arange(g, dtype=np.float32), fc).reshape(1, lanes))
    gvec = grid.astype(jnp.float32).reshape(1, g)
    bias2 = bias.astype(jnp.float32).reshape(1, o)

    return pl.pallas_call(
        lambda *refs: _tc_body(nc, fc, g, o, *refs),
        grid=(b // bb,),
        in_specs=[
            pl.BlockSpec((bb, f), lambda i: (i, 0)),
            pl.BlockSpec((1, g), lambda i: (0, 0)),
            pl.BlockSpec((1, lanes), lambda i: (0, 0)),
            pl.BlockSpec((nc * g * fc, o), lambda i: (0, 0)),
            pl.BlockSpec((2 * f, o), lambda i: (0, 0)),
            pl.BlockSpec((1, o), lambda i: (0, 0)),
        ],
        out_specs=pl.BlockSpec((bb, o), lambda i: (i, 0)),
        out_shape=jax.ShapeDtypeStruct((b, o), jnp.float32),
    )(x, gvec, glane, creo, cd, bias2)


def kernel(x, coeff, bias, grid):
    x = x.astype(jnp.float32)
    if x.ndim != 2:
        x = x.reshape(x.shape[0], -1)
    b = x.shape[0]
    b_sc = b // 2
    b_sc -= b_sc % (_BG * _CPB * _L)
    if b_sc == 0:
        return _tc_kan(x, coeff, bias, grid)
    y_tc = _tc_kan(x[:b - b_sc], coeff, bias, grid)
    y_sc = _sc_kan(x[b - b_sc:], coeff, bias, grid)
    return jnp.concatenate([y_tc, y_sc], axis=0)


# speedup vs baseline: 5.8248x; 1.4743x over previous
import functools
import jax
import jax.numpy as jnp
import numpy as np
from jax import lax
from jax.experimental import pallas as pl
from jax.experimental.pallas import tpu as pltpu
from jax.experimental.pallas import tpu_sc as plsc

_NC = 2
_NS = 16
_L = 16
_OG = 8
_BG = 4
_CPB = 4


def _sc_body(f, g, opg, bc, nbv, xc_ref, cre_ref, gvec_ref, bvec_ref,
             out_ref, cv, xbuf, yv, gv, bv):
    wid = lax.axis_index("s") * _NC + lax.axis_index("c")
    og = lax.rem(wid, _OG)
    bg = lax.div(wid, _OG)

    pltpu.sync_copy(cre_ref.at[og], cv)
    pltpu.sync_copy(gvec_ref, gv)
    pltpu.sync_copy(bvec_ref, bv)

    ghead = gv[pl.ds(0, _L)]
    gtail = gv[pl.ds(g - _L, _L)]
    gmin = ghead[0]
    invh = (g - 1.0) / jnp.full((_L,), gtail[_L - 1] - gmin, jnp.float32)
    ob = og * opg
    bias_init = tuple(
        plsc.load_gather(bv, [jnp.full((_L,), ob + j, jnp.int32)])
        for j in range(opg))

    def f_body(fi, accs):
        xv = xbuf[pl.ds(fi * bc + f_body_base[0], _L)]
        u = (xv - gmin) * invh
        it = u.astype(jnp.int32)
        ic = it + jnp.where(u > it.astype(jnp.float32), 1, 0)
        idx = jnp.clip(ic, 1, g - 1)
        i0 = idx - 1
        t = u - i0.astype(jnp.float32)
        ib0 = i0 + fi * (g * opg)
        out = []
        for j in range(opg):
            c0 = plsc.load_gather(cv, [ib0 + j * g])
            c1 = plsc.load_gather(cv, [ib0 + (j * g + 1)])
            out.append(accs[j] + (c0 + t * (c1 - c0)))
        return tuple(out)

    f_body_base = [None]

    def bvec_body(bi, carry):
        bb = bi * _L
        f_body_base[0] = bb
        accs = lax.fori_loop(0, f, f_body, bias_init)
        for j in range(opg):
            yv[j, pl.ds(bb, _L)] = accs[j]
        return carry

    def chunk_body(ci, carry):
        cg = bg * _CPB + ci
        pltpu.sync_copy(xc_ref.at[cg], xbuf)
        lax.fori_loop(0, bc // _L, bvec_body, 0)
        pltpu.sync_copy(yv, out_ref.at[cg, og])
        return carry

    lax.fori_loop(0, _CPB, chunk_body, 0)


def _sc_kan(x, coeff, bias, grid):
    b, f = x.shape
    g = grid.shape[0]
    o = coeff.shape[-1]
    opg = o // _OG
    nch = _BG * _CPB
    bc = b // nch

    xc = x.reshape(nch, bc, f).transpose(0, 2, 1).reshape(nch, f * bc)
    cre = (coeff.astype(jnp.float32).reshape(f, g, _OG, opg)
           .transpose(2, 0, 3, 1).reshape(_OG, f * opg * g))
    gvec = grid.astype(jnp.float32)
    bvec = bias.astype(jnp.float32)

    mesh = plsc.VectorSubcoreMesh(core_axis_name="c", subcore_axis_name="s")
    run = functools.partial(
        pl.kernel,
        mesh=mesh,
        compiler_params=pltpu.CompilerParams(needs_layout_passes=False),
        out_type=jax.ShapeDtypeStruct((nch, _OG, opg, bc), jnp.float32),
        scratch_types=[
            pltpu.VMEM((f * g * opg,), jnp.float32),
            pltpu.VMEM((bc * f,), jnp.float32),
            pltpu.VMEM((opg, bc), jnp.float32),
            pltpu.VMEM((g,), jnp.float32),
            pltpu.VMEM((o,), jnp.float32),
        ],
    )(functools.partial(_sc_body, f, g, opg, bc, _OG * opg))
    yblk = run(xc, cre, gvec, bvec)
    return yblk.transpose(0, 3, 1, 2).reshape(b, o)


def _tc_body(nc, fc, g, o, x_ref, gvec_ref, glane_ref, creo_ref, cd_ref,
             bias_ref, o_ref):
    x = x_ref[...]
    gmin = gvec_ref[0, 0]
    gmax = gvec_ref[0, g - 1]
    invh = (g - 1.0) / (gmax - gmin)
    u = (x - gmin) * invh
    uc = jnp.clip(u, 0.0, g - 1.0)
    d = u - uc
    dneg = jnp.minimum(d, 0.0)
    dpos = d - dneg

    glane = glane_ref[...]
    lanes = fc * g
    acc = None
    for c in range(nc):
        ucc = uc[:, c * fc:(c + 1) * fc]
        uce = pltpu.repeat(ucc, g, axis=1)
        dd = (uce - glane).astype(jnp.bfloat16)
        s = jnp.maximum(jnp.bfloat16(1.0) - jnp.abs(dd), jnp.bfloat16(0.0))
        p = jnp.dot(s, creo_ref[c * lanes:(c + 1) * lanes, :],
                    preferred_element_type=jnp.float32)
        acc = p if acc is None else acc + p

    dcat = jnp.concatenate([dneg, dpos], axis=1)
    acc = acc + jnp.dot(dcat, cd_ref[...],
                        preferred_element_type=jnp.float32)
    o_ref[...] = acc + bias_ref[...]


def _tc_kan(x, coeff, bias, grid):
    b, f = x.shape
    g = grid.shape[0]
    o = coeff.shape[-1]

    fc = 16
    nc = f // fc
    lanes = fc * g
    bb = min(b, 1024)

    creo = coeff.astype(jnp.float32).reshape(nc, fc, g, o)
    creo = creo.transpose(0, 2, 1, 3).reshape(nc * g * fc, o)
    creo = creo.astype(jnp.bfloat16)
    cd = jnp.concatenate([coeff[:, 1, :] - coeff[:, 0, :],
                          coeff[:, g - 1, :] - coeff[:, g - 2, :]],
                         axis=0).astype(jnp.float32)
    glane = jnp.asarray(
        np.repeat(np.arange(g, dtype=np.float32), fc).reshape(1, lanes))
    gvec = grid.astype(jnp.float32).reshape(1, g)
    bias2 = bias.astype(jnp.float32).reshape(1, o)

    return pl.pallas_call(
        lambda *refs: _tc_body(nc, fc, g, o, *refs),
        grid=(b // bb,),
        in_specs=[
            pl.BlockSpec((bb, f), lambda i: (i, 0)),
            pl.BlockSpec((1, g), lambda i: (0, 0)),
            pl.BlockSpec((1, lanes), lambda i: (0, 0)),
            pl.BlockSpec((nc * g * fc, o), lambda i: (0, 0)),
            pl.BlockSpec((2 * f, o), lambda i: (0, 0)),
            pl.BlockSpec((1, o), lambda i: (0, 0)),
        ],
        out_specs=pl.BlockSpec((bb, o), lambda i: (i, 0)),
        out_shape=jax.ShapeDtypeStruct((b, o), jnp.float32),
    )(x, gvec, glane, creo, cd, bias2)


def kernel(x, coeff, bias, grid):
    x = x.astype(jnp.float32)
    if x.ndim != 2:
        x = x.reshape(x.shape[0], -1)
    b = x.shape[0]
    b_sc = b // 4
    b_sc -= b_sc % (_BG * _CPB * _L)
    if b_sc == 0:
        return _tc_kan(x, coeff, bias, grid)
    y_tc = _tc_kan(x[:b - b_sc], coeff, bias, grid)
    y_sc = _sc_kan(x[b - b_sc:], coeff, bias, grid)
    return jnp.concatenate([y_tc, y_sc], axis=0)


# speedup vs baseline: 6.5437x; 1.1234x over previous
import functools
import jax
import jax.numpy as jnp
import numpy as np
from jax import lax
from jax.experimental import pallas as pl
from jax.experimental.pallas import tpu as pltpu
from jax.experimental.pallas import tpu_sc as plsc

_NC = 2
_NS = 16
_L = 16
_OG = 8
_BG = 4
_CPB = 4


def _sc_body(f, g, opg, bc, nbv, xc_ref, cre_ref, gvec_ref, bvec_ref,
             out_ref, cv, xbuf, yv, gv, bv):
    wid = lax.axis_index("s") * _NC + lax.axis_index("c")
    og = lax.rem(wid, _OG)
    bg = lax.div(wid, _OG)

    pltpu.sync_copy(cre_ref.at[og], cv)
    pltpu.sync_copy(gvec_ref, gv)
    pltpu.sync_copy(bvec_ref, bv)

    ghead = gv[pl.ds(0, _L)]
    gtail = gv[pl.ds(g - _L, _L)]
    gmin = ghead[0]
    invh = (g - 1.0) / jnp.full((_L,), gtail[_L - 1] - gmin, jnp.float32)
    ob = og * opg
    bias_init = tuple(
        plsc.load_gather(bv, [jnp.full((_L,), ob + j, jnp.int32)])
        for j in range(opg))

    def f_body(fi, accs):
        xv = xbuf[pl.ds(fi * bc + f_body_base[0], _L)]
        u = (xv - gmin) * invh
        it = u.astype(jnp.int32)
        ic = it + jnp.where(u > it.astype(jnp.float32), 1, 0)
        idx = jnp.clip(ic, 1, g - 1)
        i0 = idx - 1
        t = u - i0.astype(jnp.float32)
        ib0 = i0 + fi * (g * opg)
        out = []
        for j in range(opg):
            c0 = plsc.load_gather(cv, [ib0 + j * g])
            c1 = plsc.load_gather(cv, [ib0 + (j * g + 1)])
            out.append(accs[j] + (c0 + t * (c1 - c0)))
        return tuple(out)

    f_body_base = [None]

    def bvec_body(bi, carry):
        bb = bi * _L
        f_body_base[0] = bb
        accs = lax.fori_loop(0, f, f_body, bias_init)
        for j in range(opg):
            yv[j, pl.ds(bb, _L)] = accs[j]
        return carry

    def chunk_body(ci, carry):
        cg = bg * _CPB + ci
        pltpu.sync_copy(xc_ref.at[cg], xbuf)
        lax.fori_loop(0, bc // _L, bvec_body, 0)
        pltpu.sync_copy(yv, out_ref.at[cg, og])
        return carry

    lax.fori_loop(0, _CPB, chunk_body, 0)


def _sc_kan(x, coeff, bias, grid):
    b, f = x.shape
    g = grid.shape[0]
    o = coeff.shape[-1]
    opg = o // _OG
    nch = _BG * _CPB
    bc = b // nch

    xc = x.reshape(nch, bc, f).transpose(0, 2, 1).reshape(nch, f * bc)
    cre = (coeff.astype(jnp.float32).reshape(f, g, _OG, opg)
           .transpose(2, 0, 3, 1).reshape(_OG, f * opg * g))
    gvec = grid.astype(jnp.float32)
    bvec = bias.astype(jnp.float32)

    mesh = plsc.VectorSubcoreMesh(core_axis_name="c", subcore_axis_name="s")
    run = functools.partial(
        pl.kernel,
        mesh=mesh,
        compiler_params=pltpu.CompilerParams(needs_layout_passes=False),
        out_type=jax.ShapeDtypeStruct((nch, _OG, opg, bc), jnp.float32),
        scratch_types=[
            pltpu.VMEM((f * g * opg,), jnp.float32),
            pltpu.VMEM((bc * f,), jnp.float32),
            pltpu.VMEM((opg, bc), jnp.float32),
            pltpu.VMEM((g,), jnp.float32),
            pltpu.VMEM((o,), jnp.float32),
        ],
    )(functools.partial(_sc_body, f, g, opg, bc, _OG * opg))
    yblk = run(xc, cre, gvec, bvec)
    return yblk.transpose(0, 3, 1, 2).reshape(b, o)


def _tc_body(nc, fc, g, o, x_ref, gvec_ref, glane_ref, creo_ref, cd_ref,
             bias_ref, o_ref):
    x = x_ref[...]
    gmin = gvec_ref[0, 0]
    gmax = gvec_ref[0, g - 1]
    invh = (g - 1.0) / (gmax - gmin)
    u = (x - gmin) * invh
    uc = jnp.clip(u, 0.0, g - 1.0)
    d = u - uc
    dneg = jnp.minimum(d, 0.0)
    dpos = d - dneg

    glane = glane_ref[...]
    lanes = fc * g
    acc = None
    for c in range(nc):
        ucc = uc[:, c * fc:(c + 1) * fc]
        uce = pltpu.repeat(ucc, g, axis=1)
        dd = (uce - glane).astype(jnp.bfloat16)
        s = jnp.maximum(jnp.bfloat16(1.0) - jnp.abs(dd), jnp.bfloat16(0.0))
        p = jnp.dot(s, creo_ref[c * lanes:(c + 1) * lanes, :],
                    preferred_element_type=jnp.float32)
        acc = p if acc is None else acc + p

    dcat = jnp.concatenate([dneg, dpos], axis=1)
    acc = acc + jnp.dot(dcat, cd_ref[...],
                        preferred_element_type=jnp.float32)
    o_ref[...] = acc + bias_ref[...]


def _tc_kan(x, coeff, bias, grid):
    b, f = x.shape
    g = grid.shape[0]
    o = coeff.shape[-1]

    fc = 16
    nc = f // fc
    lanes = fc * g
    bb = min(b, 1024)
    while b % bb:
        bb //= 2

    creo = coeff.astype(jnp.float32).reshape(nc, fc, g, o)
    creo = creo.transpose(0, 2, 1, 3).reshape(nc * g * fc, o)
    creo = creo.astype(jnp.bfloat16)
    cd = jnp.concatenate([coeff[:, 1, :] - coeff[:, 0, :],
                          coeff[:, g - 1, :] - coeff[:, g - 2, :]],
                         axis=0).astype(jnp.float32)
    glane = jnp.asarray(
        np.repeat(np.arange(g, dtype=np.float32), fc).reshape(1, lanes))
    gvec = grid.astype(jnp.float32).reshape(1, g)
    bias2 = bias.astype(jnp.float32).reshape(1, o)

    return pl.pallas_call(
        lambda *refs: _tc_body(nc, fc, g, o, *refs),
        grid=(b // bb,),
        in_specs=[
            pl.BlockSpec((bb, f), lambda i: (i, 0)),
            pl.BlockSpec((1, g), lambda i: (0, 0)),
            pl.BlockSpec((1, lanes), lambda i: (0, 0)),
            pl.BlockSpec((nc * g * fc, o), lambda i: (0, 0)),
            pl.BlockSpec((2 * f, o), lambda i: (0, 0)),
            pl.BlockSpec((1, o), lambda i: (0, 0)),
        ],
        out_specs=pl.BlockSpec((bb, o), lambda i: (i, 0)),
        out_shape=jax.ShapeDtypeStruct((b, o), jnp.float32),
    )(x, gvec, glane, creo, cd, bias2)


def kernel(x, coeff, bias, grid):
    x = x.astype(jnp.float32)
    if x.ndim != 2:
        x = x.reshape(x.shape[0], -1)
    b = x.shape[0]
    b_sc = b // 8
    b_sc -= b_sc % (_BG * _CPB * _L)
    if b_sc == 0:
        return _tc_kan(x, coeff, bias, grid)
    y_tc = _tc_kan(x[:b - b_sc], coeff, bias, grid)
    y_sc = _sc_kan(x[b - b_sc:], coeff, bias, grid)
    return jnp.concatenate([y_tc, y_sc], axis=0)
